# baseline (speedup 1.0000x reference)
import functools
import math

import jax
import jax.numpy as jnp
from jax.experimental import pallas as pl

B, S, D, H, K, RES, SAMPLE = 2, 2048, 768, 12, 16, 32, 128
HD = D // H
FF = 4 * D
N = S * K

RQ = 512
RD = 512
RF = 256


def _qkv_kernel(x_ref, wq, bq, wk, bk, wv, bv, q_out, k_out, v_out):
    xb = x_ref[...]
    q_out[...] = jax.lax.dot_general(
        xb, wq[...], (((1,), (0,)), ((), ())),
        preferred_element_type=jnp.float32) + bq[...]
    k_out[...] = jax.lax.dot_general(
        xb, wk[...], (((1,), (0,)), ((), ())),
        preferred_element_type=jnp.float32) + bk[...]
    v_out[...] = jax.lax.dot_general(
        xb, wv[...], (((1,), (0,)), ((), ())),
        preferred_element_type=jnp.float32) + bv[...]


def _attn_kernel(q_ref, k_ref, v_ref, o_ref):
    qb = q_ref[0, 0]
    kb = k_ref[0, 0]
    vb = v_ref[0, 0]
    s = jax.lax.dot_general(qb, kb, (((1,), (1,)), ((), ())),
                            preferred_element_type=jnp.float32)
    s = s * (1.0 / math.sqrt(HD))
    m = jnp.max(s, axis=-1, keepdims=True)
    e = jnp.exp(s - m)
    p = e / jnp.sum(e, axis=-1, keepdims=True)
    o_ref[0, 0] = jax.lax.dot_general(p, vb, (((1,), (0,)), ((), ())),
                                      preferred_element_type=jnp.float32)


def _dist_kernel(x_ref, xs_ref, out_ref):
    i = pl.program_id(1)
    xb = x_ref[0]
    xsb = xs_ref[0]
    xn = jnp.sum(xb * xb, axis=-1, keepdims=True)
    xsn = jnp.sum(xsb * xsb, axis=-1)[None, :]
    g = jax.lax.dot_general(xb, xsb, (((1,), (1,)), ((), ())),
                            preferred_element_type=jnp.float32)
    d2 = xn + xsn - 2.0 * g

    lane = jax.lax.broadcasted_iota(jnp.int32, (RD, SAMPLE), 1)
    work = d2
    s_acc = jnp.float32(0.0)
    ss_acc = jnp.float32(0.0)
    minv = jnp.float32(jnp.inf)
    maxv = jnp.float32(-jnp.inf)
    for it in range(K):
        m = jnp.min(work, axis=1, keepdims=True)
        dv = jnp.sqrt(jnp.maximum(m, 0.0))
        s_acc = s_acc + jnp.sum(dv)
        ss_acc = ss_acc + jnp.sum(dv * dv)
        if it == 0:
            minv = jnp.min(dv)
        if it == K - 1:
            maxv = jnp.max(dv)
        if it < K - 1:
            cand = jnp.where(work == m, lane, SAMPLE)
            j = jnp.min(cand, axis=1, keepdims=True)
            work = jnp.where(lane == j, jnp.inf, work)

    lout = jax.lax.broadcasted_iota(jnp.int32, (1, 128), 1)
    cur = jnp.where(lout == 0, s_acc,
          jnp.where(lout == 1, ss_acc,
          jnp.where(lout == 2, minv,
          jnp.where(lout == 3, maxv, 0.0))))

    @pl.when(i == 0)
    def _():
        init = jnp.where(lout == 2, jnp.inf,
               jnp.where(lout == 3, -jnp.inf, 0.0))
        out_ref[0] = init

    prev = out_ref[0]
    out_ref[0] = jnp.where(lout < 2, prev + cur,
                 jnp.where(lout == 2, jnp.minimum(prev, cur),
                 jnp.where(lout == 3, jnp.maximum(prev, cur), 0.0)))


def _topo_kernel(part_ref, s1w, s1b, s2w, s2b, p0w, p0b, p1w, p1b,
                 tw, tb, gate_ref, out_ref):
    part = part_ref[:, 0, :]
    sumv = part[:, 0:1]
    sumsq = part[:, 1:2]
    mn = part[:, 2:3]
    mx = part[:, 3:4]
    mean = sumv / N
    var = (sumsq - N * mean * mean) / (N - 1)
    std = jnp.sqrt(jnp.maximum(var, 0.0))
    z = jnp.zeros_like(mean)

    l8 = jax.lax.broadcasted_iota(jnp.int32, (B, 8), 1)

    def build(c0, c1, c2, c3, c4, c5):
        return jnp.where(l8 == 0, c0,
               jnp.where(l8 == 1, c1,
               jnp.where(l8 == 2, c2,
               jnp.where(l8 == 3, c3,
               jnp.where(l8 == 4, c4,
               jnp.where(l8 == 5, c5, z))))))

    stats0 = build(mean, std, mn, mx, mean / 2, std / 2)
    stats1 = build(mean * 0.7, std * 0.7, mean * 0.3, mean * 1.2,
                   mean * 0.5, std * 0.3)

    def landscape(st, pw, pb):
        h = jax.lax.dot_general(st, s1w[...], (((1,), (0,)), ((), ())),
                                preferred_element_type=jnp.float32) + s1b[...]
        h = jnp.maximum(h, 0.0)
        l = jax.lax.dot_general(h, s2w[...], (((1,), (0,)), ((), ())),
                                preferred_element_type=jnp.float32) + s2b[...]
        return jax.lax.dot_general(l, pw[...], (((1,), (0,)), ((), ())),
                                   preferred_element_type=jnp.float32) + pb[...]

    l0 = landscape(stats0, p0w, p0b)
    l1 = landscape(stats1, p1w, p1b)
    lm = 0.5 * (l0 + l1)
    topo = jax.lax.dot_general(lm, tw[...], (((1,), (0,)), ((), ())),
                               preferred_element_type=jnp.float32) + tb[...]
    out_ref[...] = (gate_ref[0, 0] * topo)[:, None, :]


def _ffn_kernel(x_ref, ctx_ref, topo_ref, wo, bo, ln1g, ln1b,
                w1, b1, w2, b2, ln2g, ln2b, out_ref):
    xb = x_ref[0]
    cb = ctx_ref[0]
    attn = jax.lax.dot_general(cb, wo[...], (((1,), (0,)), ((), ())),
                               preferred_element_type=jnp.float32) + bo[...]
    attn = attn + topo_ref[0]
    pre = xb + attn
    mu = jnp.mean(pre, axis=-1, keepdims=True)
    var = jnp.mean((pre - mu) ** 2, axis=-1, keepdims=True)
    h = (pre - mu) / jnp.sqrt(var + 1e-5) * ln1g[...] + ln1b[...]
    f = jax.lax.dot_general(h, w1[...], (((1,), (0,)), ((), ())),
                            preferred_element_type=jnp.float32) + b1[...]
    f = jax.nn.gelu(f)
    f = jax.lax.dot_general(f, w2[...], (((1,), (0,)), ((), ())),
                            preferred_element_type=jnp.float32) + b2[...]
    pre2 = h + f
    mu2 = jnp.mean(pre2, axis=-1, keepdims=True)
    var2 = jnp.mean((pre2 - mu2) ** 2, axis=-1, keepdims=True)
    out_ref[0] = (pre2 - mu2) / jnp.sqrt(var2 + 1e-5) * ln2g[...] + ln2b[...]


def _row2d(a):
    return a.reshape(1, -1)


@jax.jit
def kernel(x, params, sample_idx):
    p = params
    xs = jnp.take(x, sample_idx, axis=1)

    xf = x.reshape(B * S, D)
    nrb = (B * S) // RQ
    qkv = pl.pallas_call(
        _qkv_kernel,
        grid=(nrb,),
        in_specs=[
            pl.BlockSpec((RQ, D), lambda i: (i, 0)),
            pl.BlockSpec((D, D), lambda i: (0, 0)),
            pl.BlockSpec((1, D), lambda i: (0, 0)),
            pl.BlockSpec((D, D), lambda i: (0, 0)),
            pl.BlockSpec((1, D), lambda i: (0, 0)),
            pl.BlockSpec((D, D), lambda i: (0, 0)),
            pl.BlockSpec((1, D), lambda i: (0, 0)),
        ],
        out_specs=[pl.BlockSpec((RQ, D), lambda i: (i, 0))] * 3,
        out_shape=[jax.ShapeDtypeStruct((B * S, D), jnp.float32)] * 3,
    )(xf, p["q"]["w"], _row2d(p["q"]["b"]),
      p["k"]["w"], _row2d(p["k"]["b"]),
      p["v"]["w"], _row2d(p["v"]["b"]))
    q4 = qkv[0].reshape(B, S, H, HD).transpose(0, 2, 1, 3)
    k4 = qkv[1].reshape(B, S, H, HD).transpose(0, 2, 1, 3)
    v4 = qkv[2].reshape(B, S, H, HD).transpose(0, 2, 1, 3)

    ctx4 = pl.pallas_call(
        _attn_kernel,
        grid=(B, H, S // RQ),
        in_specs=[
            pl.BlockSpec((1, 1, RQ, HD), lambda b, h, i: (b, h, i, 0)),
            pl.BlockSpec((1, 1, S, HD), lambda b, h, i: (b, h, 0, 0)),
            pl.BlockSpec((1, 1, S, HD), lambda b, h, i: (b, h, 0, 0)),
        ],
        out_specs=pl.BlockSpec((1, 1, RQ, HD), lambda b, h, i: (b, h, i, 0)),
        out_shape=jax.ShapeDtypeStruct((B, H, S, HD), jnp.float32),
    )(q4, k4, v4)
    ctx = ctx4.transpose(0, 2, 1, 3).reshape(B, S, D)

    parts = pl.pallas_call(
        _dist_kernel,
        grid=(B, S // RD),
        in_specs=[
            pl.BlockSpec((1, RD, D), lambda b, i: (b, i, 0)),
            pl.BlockSpec((1, SAMPLE, D), lambda b, i: (b, 0, 0)),
        ],
        out_specs=pl.BlockSpec((1, 1, 128), lambda b, i: (b, 0, 0)),
        out_shape=jax.ShapeDtypeStruct((B, 1, 128), jnp.float32),
    )(x, xs)

    gate = p["topo_gate"].reshape(1, 1)
    topo = pl.pallas_call(
        _topo_kernel,
        in_specs=[pl.BlockSpec(memory_space=pl.ANY)] * 0 + [
            pl.BlockSpec((B, 1, 128), lambda: (0, 0, 0)),
            pl.BlockSpec((8, D // 4), lambda: (0, 0)),
            pl.BlockSpec((1, D // 4), lambda: (0, 0)),
            pl.BlockSpec((D // 4, RES), lambda: (0, 0)),
            pl.BlockSpec((1, RES), lambda: (0, 0)),
            pl.BlockSpec((RES, RES), lambda: (0, 0)),
            pl.BlockSpec((1, RES), lambda: (0, 0)),
            pl.BlockSpec((RES, RES), lambda: (0, 0)),
            pl.BlockSpec((1, RES), lambda: (0, 0)),
            pl.BlockSpec((RES, D), lambda: (0, 0)),
            pl.BlockSpec((1, D), lambda: (0, 0)),
            pl.BlockSpec((1, 1), lambda: (0, 0)),
        ],
        out_specs=pl.BlockSpec((B, 1, D), lambda: (0, 0, 0)),
        out_shape=jax.ShapeDtypeStruct((B, 1, D), jnp.float32),
    )(parts, p["stats1"]["w"], _row2d(p["stats1"]["b"]),
      p["stats2"]["w"], _row2d(p["stats2"]["b"]),
      p["proc0"]["w"], _row2d(p["proc0"]["b"]),
      p["proc1"]["w"], _row2d(p["proc1"]["b"]),
      p["topo_proj"]["w"], _row2d(p["topo_proj"]["b"]), gate)

    out = pl.pallas_call(
        _ffn_kernel,
        grid=(B, S // RF),
        in_specs=[
            pl.BlockSpec((1, RF, D), lambda b, i: (b, i, 0)),
            pl.BlockSpec((1, RF, D), lambda b, i: (b, i, 0)),
            pl.BlockSpec((1, 1, D), lambda b, i: (b, 0, 0)),
            pl.BlockSpec((D, D), lambda b, i: (0, 0)),
            pl.BlockSpec((1, D), lambda b, i: (0, 0)),
            pl.BlockSpec((1, D), lambda b, i: (0, 0)),
            pl.BlockSpec((1, D), lambda b, i: (0, 0)),
            pl.BlockSpec((D, FF), lambda b, i: (0, 0)),
            pl.BlockSpec((1, FF), lambda b, i: (0, 0)),
            pl.BlockSpec((FF, D), lambda b, i: (0, 0)),
            pl.BlockSpec((1, D), lambda b, i: (0, 0)),
            pl.BlockSpec((1, D), lambda b, i: (0, 0)),
            pl.BlockSpec((1, D), lambda b, i: (0, 0)),
        ],
        out_specs=pl.BlockSpec((1, RF, D), lambda b, i: (b, i, 0)),
        out_shape=jax.ShapeDtypeStruct((B, S, D), jnp.float32),
    )(x, ctx, topo,
      p["o"]["w"], _row2d(p["o"]["b"]),
      _row2d(p["ln1_g"]), _row2d(p["ln1_b"]),
      p["ffn1"]["w"], _row2d(p["ffn1"]["b"]),
      p["ffn2"]["w"], _row2d(p["ffn2"]["b"]),
      _row2d(p["ln2_g"]), _row2d(p["ln2_b"]))
    return out


# speedup vs baseline: 1.6535x; 1.6535x over previous
import math

import jax
import jax.numpy as jnp
from jax.experimental import pallas as pl
from jax.experimental.pallas import tpu as pltpu
from jax.experimental.pallas import tpu_sc as plsc

B, S, D, H, K, RES, SAMPLE = 2, 2048, 768, 12, 16, 32, 128
HD = D // H
FF = 4 * D
N = S * K

RQ = 512
RF = 256


def _dot(a, b):
    return jax.lax.dot_general(a, b, (((1,), (0,)), ((), ())),
                               preferred_element_type=jnp.float32)


def _dot_t(a, b):
    return jax.lax.dot_general(a, b, (((1,), (1,)), ((), ())),
                               preferred_element_type=jnp.float32)


def _bf(a):
    return a.astype(jnp.bfloat16)


def _attn_kernel(xq_ref, xkv_ref, xs_ref, wq, bq, wk, bk, wv, bv,
                 ctx_ref, parts_ref, k_s, v_s):
    i = pl.program_id(1)

    @pl.when(i == 0)
    def _():
        xkv = _bf(xkv_ref[0])
        k_s[...] = _bf(_dot(xkv, wk[...]) + bk[...])
        v_s[...] = _bf(_dot(xkv, wv[...]) + bv[...])

    xq = xq_ref[0]
    q = _bf(_dot(_bf(xq), wq[...]) + bq[...])

    for h in range(H):
        sl = slice(h * HD, (h + 1) * HD)
        s = _dot_t(q[:, sl], k_s[:, sl]) * (1.0 / math.sqrt(HD))
        m = jnp.max(s, axis=-1, keepdims=True)
        e = jnp.exp(s - m)
        p = _bf(e / jnp.sum(e, axis=-1, keepdims=True))
        ctx_ref[0, :, sl] = _dot(p, v_s[:, sl])

    xsb = xs_ref[0]
    xn = jnp.sum(xq * xq, axis=-1, keepdims=True)
    xsn = jnp.sum(xsb * xsb, axis=-1)[None, :]
    d2 = xn + xsn - 2.0 * _dot_t(xq, xsb)

    lane = jax.lax.broadcasted_iota(jnp.int32, (RQ, SAMPLE), 1)
    work = d2
    s_acc = jnp.float32(0.0)
    ss_acc = jnp.float32(0.0)
    minv = jnp.float32(jnp.inf)
    maxv = jnp.float32(-jnp.inf)
    for it in range(K):
        m = jnp.min(work, axis=1, keepdims=True)
        dv = jnp.sqrt(jnp.maximum(m, 0.0))
        s_acc = s_acc + jnp.sum(dv)
        ss_acc = ss_acc + jnp.sum(dv * dv)
        if it == 0:
            minv = jnp.min(dv)
        if it == K - 1:
            maxv = jnp.max(dv)
        if it < K - 1:
            cand = jnp.where(work == m, lane, SAMPLE)
            j = jnp.min(cand, axis=1, keepdims=True)
            work = jnp.where(lane == j, jnp.inf, work)

    lout = jax.lax.broadcasted_iota(jnp.int32, (1, 128), 1)
    cur = jnp.where(lout == 0, s_acc,
          jnp.where(lout == 1, ss_acc,
          jnp.where(lout == 2, minv,
          jnp.where(lout == 3, maxv, 0.0))))

    @pl.when(i == 0)
    def _():
        parts_ref[0] = jnp.where(lout == 2, jnp.inf,
                       jnp.where(lout == 3, -jnp.inf, 0.0))

    prev = parts_ref[0]
    parts_ref[0] = jnp.where(lout < 2, prev + cur,
                   jnp.where(lout == 2, jnp.minimum(prev, cur),
                   jnp.where(lout == 3, jnp.maximum(prev, cur), 0.0)))


def _ffn_kernel(x_ref, ctx_ref, parts_ref, gate_ref, wo, bo,
                s1w, s1b, s2w, s2b, p0w, p0b, p1w, p1b, tw, tb,
                ln1g, ln1b, w1, b1, w2, b2, ln2g, ln2b, out_ref):
    part = parts_ref[0]
    sumv = part[:, 0:1]
    sumsq = part[:, 1:2]
    mn = part[:, 2:3]
    mx = part[:, 3:4]
    mean = sumv / N
    var = (sumsq - N * mean * mean) / (N - 1)
    std = jnp.sqrt(jnp.maximum(var, 0.0))
    z = jnp.zeros_like(mean)

    l8 = jax.lax.broadcasted_iota(jnp.int32, (1, 8), 1)

    def build(c0, c1, c2, c3, c4, c5):
        return jnp.where(l8 == 0, c0,
               jnp.where(l8 == 1, c1,
               jnp.where(l8 == 2, c2,
               jnp.where(l8 == 3, c3,
               jnp.where(l8 == 4, c4,
               jnp.where(l8 == 5, c5, z))))))

    stats0 = build(mean, std, mn, mx, mean / 2, std / 2)
    stats1 = build(mean * 0.7, std * 0.7, mean * 0.3, mean * 1.2,
                   mean * 0.5, std * 0.3)

    def landscape(st, pw, pb):
        hh = jnp.maximum(_dot(st, s1w[...]) + s1b[...], 0.0)
        ll = _dot(hh, s2w[...]) + s2b[...]
        return _dot(ll, pw[...]) + pb[...]

    lm = 0.5 * (landscape(stats0, p0w, p0b) + landscape(stats1, p1w, p1b))
    topo = _dot(lm, tw[...]) + tb[...]
    topo = gate_ref[0, 0] * topo

    xb = x_ref[0]
    cb = _bf(ctx_ref[0])
    attn = _dot(cb, wo[...]) + bo[...] + topo
    pre = xb + attn
    mu = jnp.mean(pre, axis=-1, keepdims=True)
    var1 = jnp.mean((pre - mu) ** 2, axis=-1, keepdims=True)
    h = (pre - mu) / jnp.sqrt(var1 + 1e-5) * ln1g[...] + ln1b[...]
    f = jax.nn.gelu(_dot(_bf(h), w1[...]) + b1[...])
    f = _dot(_bf(f), w2[...]) + b2[...]
    pre2 = h + f
    mu2 = jnp.mean(pre2, axis=-1, keepdims=True)
    var2 = jnp.mean((pre2 - mu2) ** 2, axis=-1, keepdims=True)
    out_ref[0] = (pre2 - mu2) / jnp.sqrt(var2 + 1e-5) * ln2g[...] + ln2b[...]


def _row2d(a):
    return a.reshape(1, -1)


_GW = (B * SAMPLE) // 32


def _sc_gather(x2d, idx2d):
    mesh = plsc.VectorSubcoreMesh(core_axis_name="core",
                                  subcore_axis_name="subcore")

    @pl.kernel(out_type=jax.ShapeDtypeStruct((B * SAMPLE, D), jnp.float32),
               mesh=mesh)
    def kern(x_hbm, i_hbm, o_hbm):
        def body(i_vmem, o_vmem):
            pltpu.sync_copy(x_hbm.at[i_vmem.at[0]], o_vmem)

        pltpu.emit_pipeline(
            body,
            grid=((B * SAMPLE) // _GW,),
            in_specs=[pl.BlockSpec((1, _GW), lambda i: (0, i))],
            out_specs=[pl.BlockSpec((_GW, D), lambda i: (i, 0))],
            core_axis_name=("core", "subcore"),
            dimension_semantics=(pltpu.PARALLEL,),
        )(i_hbm, o_hbm)

    return kern(x2d, idx2d)


@jax.jit
def kernel(x, params, sample_idx):
    p = params
    xs = jnp.take(x, sample_idx, axis=1)

    ctx, parts = pl.pallas_call(
        _attn_kernel,
        grid=(B, S // RQ),
        in_specs=[
            pl.BlockSpec((1, RQ, D), lambda b, i: (b, i, 0)),
            pl.BlockSpec((1, S, D), lambda b, i: (b, 0, 0)),
            pl.BlockSpec((1, SAMPLE, D), lambda b, i: (b, 0, 0)),
            pl.BlockSpec((D, D), lambda b, i: (0, 0)),
            pl.BlockSpec((1, D), lambda b, i: (0, 0)),
            pl.BlockSpec((D, D), lambda b, i: (0, 0)),
            pl.BlockSpec((1, D), lambda b, i: (0, 0)),
            pl.BlockSpec((D, D), lambda b, i: (0, 0)),
            pl.BlockSpec((1, D), lambda b, i: (0, 0)),
        ],
        out_specs=[
            pl.BlockSpec((1, RQ, D), lambda b, i: (b, i, 0)),
            pl.BlockSpec((1, 1, 128), lambda b, i: (b, 0, 0)),
        ],
        out_shape=[
            jax.ShapeDtypeStruct((B, S, D), jnp.float32),
            jax.ShapeDtypeStruct((B, 1, 128), jnp.float32),
        ],
        scratch_shapes=[
            pltpu.VMEM((S, D), jnp.bfloat16),
            pltpu.VMEM((S, D), jnp.bfloat16),
        ],
    )(x, x, xs,
      _bf(p["q"]["w"]), _row2d(p["q"]["b"]),
      _bf(p["k"]["w"]), _row2d(p["k"]["b"]),
      _bf(p["v"]["w"]), _row2d(p["v"]["b"]))

    gate = p["topo_gate"].reshape(1, 1)
    out = pl.pallas_call(
        _ffn_kernel,
        grid=(B, S // RF),
        in_specs=[
            pl.BlockSpec((1, RF, D), lambda b, i: (b, i, 0)),
            pl.BlockSpec((1, RF, D), lambda b, i: (b, i, 0)),
            pl.BlockSpec((1, 1, 128), lambda b, i: (b, 0, 0)),
            pl.BlockSpec((1, 1), lambda b, i: (0, 0)),
            pl.BlockSpec((D, D), lambda b, i: (0, 0)),
            pl.BlockSpec((1, D), lambda b, i: (0, 0)),
            pl.BlockSpec((8, D // 4), lambda b, i: (0, 0)),
            pl.BlockSpec((1, D // 4), lambda b, i: (0, 0)),
            pl.BlockSpec((D // 4, RES), lambda b, i: (0, 0)),
            pl.BlockSpec((1, RES), lambda b, i: (0, 0)),
            pl.BlockSpec((RES, RES), lambda b, i: (0, 0)),
            pl.BlockSpec((1, RES), lambda b, i: (0, 0)),
            pl.BlockSpec((RES, RES), lambda b, i: (0, 0)),
            pl.BlockSpec((1, RES), lambda b, i: (0, 0)),
            pl.BlockSpec((RES, D), lambda b, i: (0, 0)),
            pl.BlockSpec((1, D), lambda b, i: (0, 0)),
            pl.BlockSpec((1, D), lambda b, i: (0, 0)),
            pl.BlockSpec((1, D), lambda b, i: (0, 0)),
            pl.BlockSpec((D, FF), lambda b, i: (0, 0)),
            pl.BlockSpec((1, FF), lambda b, i: (0, 0)),
            pl.BlockSpec((FF, D), lambda b, i: (0, 0)),
            pl.BlockSpec((1, D), lambda b, i: (0, 0)),
            pl.BlockSpec((1, D), lambda b, i: (0, 0)),
            pl.BlockSpec((1, D), lambda b, i: (0, 0)),
        ],
        out_specs=pl.BlockSpec((1, RF, D), lambda b, i: (b, i, 0)),
        out_shape=jax.ShapeDtypeStruct((B, S, D), jnp.float32),
    )(x, ctx, parts, gate,
      _bf(p["o"]["w"]), _row2d(p["o"]["b"]),
      p["stats1"]["w"], _row2d(p["stats1"]["b"]),
      p["stats2"]["w"], _row2d(p["stats2"]["b"]),
      p["proc0"]["w"], _row2d(p["proc0"]["b"]),
      p["proc1"]["w"], _row2d(p["proc1"]["b"]),
      p["topo_proj"]["w"], _row2d(p["topo_proj"]["b"]),
      _row2d(p["ln1_g"]), _row2d(p["ln1_b"]),
      _bf(p["ffn1"]["w"]), _row2d(p["ffn1"]["b"]),
      _bf(p["ffn2"]["w"]), _row2d(p["ffn2"]["b"]),
      _row2d(p["ln2_g"]), _row2d(p["ln2_b"]))
    return out


# speedup vs baseline: 1.8902x; 1.1432x over previous
import math

import jax
import jax.numpy as jnp
from jax.experimental import pallas as pl
from jax.experimental.pallas import tpu as pltpu
from jax.experimental.pallas import tpu_sc as plsc

B, S, D, H, K, RES, SAMPLE = 2, 2048, 768, 12, 16, 32, 128
HD = D // H
FF = 4 * D
N = S * K

RQ = 512
RF = 256


def _dot(a, b):
    return jax.lax.dot_general(a, b, (((1,), (0,)), ((), ())),
                               preferred_element_type=jnp.float32)


def _dot_t(a, b):
    return jax.lax.dot_general(a, b, (((1,), (1,)), ((), ())),
                               preferred_element_type=jnp.float32)


def _bf(a):
    return a.astype(jnp.bfloat16)


def _attn_kernel(xq_ref, xkv_ref, xs_ref, wq, bq, wk, bk, wv, bv,
                 ctx_ref, parts_ref, k_s, v_s):
    i = pl.program_id(1)

    @pl.when(i == 0)
    def _():
        xkv = xkv_ref[0]
        k_s[...] = _dot(xkv, wk[...]) + bk[...]
        v_s[...] = _dot(xkv, wv[...]) + bv[...]

    xq = xq_ref[0]
    qs = (_dot(xq, wq[...]) + bq[...]) * (1.0 / math.sqrt(HD))

    for h in range(H):
        sl = slice(h * HD, (h + 1) * HD)
        s = _dot_t(qs[:, sl], k_s[:, sl])
        m = jnp.max(s, axis=-1, keepdims=True)
        e = jnp.exp(s - m)
        r = 1.0 / jnp.sum(e, axis=-1, keepdims=True)
        ctx_ref[0, :, sl] = _dot(e, v_s[:, sl]) * r

    xsb = xs_ref[0]
    xn = jnp.sum(xq * xq, axis=-1, keepdims=True)
    xsn = jnp.sum(xsb * xsb, axis=-1)[None, :]
    d2 = xn + xsn - 2.0 * _dot_t(xq, xsb)

    lane = jax.lax.broadcasted_iota(jnp.int32, (RQ, SAMPLE), 1)
    work = d2
    s_acc = jnp.float32(0.0)
    ss_acc = jnp.float32(0.0)
    minv = jnp.float32(jnp.inf)
    maxv = jnp.float32(-jnp.inf)
    for it in range(K):
        m = jnp.min(work, axis=1, keepdims=True)
        dv = jnp.sqrt(jnp.maximum(m, 0.0))
        s_acc = s_acc + jnp.sum(dv)
        ss_acc = ss_acc + jnp.sum(dv * dv)
        if it == 0:
            minv = jnp.min(dv)
        if it == K - 1:
            maxv = jnp.max(dv)
        if it < K - 1:
            cand = jnp.where(work == m, lane, SAMPLE)
            j = jnp.min(cand, axis=1, keepdims=True)
            work = jnp.where(lane == j, jnp.inf, work)

    lout = jax.lax.broadcasted_iota(jnp.int32, (1, 128), 1)
    cur = jnp.where(lout == 0, s_acc,
          jnp.where(lout == 1, ss_acc,
          jnp.where(lout == 2, minv,
          jnp.where(lout == 3, maxv, 0.0))))

    @pl.when(i == 0)
    def _():
        parts_ref[0] = jnp.where(lout == 2, jnp.inf,
                       jnp.where(lout == 3, -jnp.inf, 0.0))

    prev = parts_ref[0]
    parts_ref[0] = jnp.where(lout < 2, prev + cur,
                   jnp.where(lout == 2, jnp.minimum(prev, cur),
                   jnp.where(lout == 3, jnp.maximum(prev, cur), 0.0)))


def _ffn_kernel(x_ref, ctx_ref, parts_ref, gate_ref, wo, bo,
                s1w, s1b, s2w, s2b, p0w, p0b, p1w, p1b, tw, tb,
                ln1g, ln1b, w1, b1, w2, b2, ln2g, ln2b, out_ref, topo_s):
    i = pl.program_id(1)

    @pl.when(i == 0)
    def _():
        part = parts_ref[0]
        sumv = part[:, 0:1]
        sumsq = part[:, 1:2]
        mn = part[:, 2:3]
        mx = part[:, 3:4]
        mean = sumv / N
        var = (sumsq - N * mean * mean) / (N - 1)
        std = jnp.sqrt(jnp.maximum(var, 0.0))
        z = jnp.zeros_like(mean)

        l8 = jax.lax.broadcasted_iota(jnp.int32, (1, 8), 1)

        def build(c0, c1, c2, c3, c4, c5):
            return jnp.where(l8 == 0, c0,
                   jnp.where(l8 == 1, c1,
                   jnp.where(l8 == 2, c2,
                   jnp.where(l8 == 3, c3,
                   jnp.where(l8 == 4, c4,
                   jnp.where(l8 == 5, c5, z))))))

        stats0 = build(mean, std, mn, mx, mean / 2, std / 2)
        stats1 = build(mean * 0.7, std * 0.7, mean * 0.3, mean * 1.2,
                       mean * 0.5, std * 0.3)

        def landscape(st, pw, pb):
            hh = jnp.maximum(_dot(st, s1w[...]) + s1b[...], 0.0)
            ll = _dot(hh, s2w[...]) + s2b[...]
            return _dot(ll, pw[...]) + pb[...]

        lm = 0.5 * (landscape(stats0, p0w, p0b) + landscape(stats1, p1w, p1b))
        topo_s[...] = gate_ref[0, 0] * (_dot(lm, tw[...]) + tb[...])

    xb = x_ref[0]
    cb = ctx_ref[0]
    attn = _dot(cb, wo[...]) + bo[...] + topo_s[...]
    pre = xb + attn
    mu = jnp.mean(pre, axis=-1, keepdims=True)
    var1 = jnp.mean((pre - mu) ** 2, axis=-1, keepdims=True)
    h = (pre - mu) / jnp.sqrt(var1 + 1e-5) * ln1g[...] + ln1b[...]
    f = jax.nn.gelu(_dot(h, w1[...]) + b1[...])
    f = _dot(f, w2[...]) + b2[...]
    pre2 = h + f
    mu2 = jnp.mean(pre2, axis=-1, keepdims=True)
    var2 = jnp.mean((pre2 - mu2) ** 2, axis=-1, keepdims=True)
    out_ref[0] = (pre2 - mu2) / jnp.sqrt(var2 + 1e-5) * ln2g[...] + ln2b[...]


def _row2d(a):
    return a.reshape(1, -1)


_GW = 128
_CHUNKS = D // 128
_NIDX = B * SAMPLE * _CHUNKS


def _sc_gather(x2d, idx2d):
    mesh = plsc.VectorSubcoreMesh(core_axis_name="core",
                                  subcore_axis_name="subcore")

    @pl.kernel(out_type=jax.ShapeDtypeStruct((_NIDX, 128), jnp.float32),
               mesh=mesh)
    def kern(x_hbm, i_hbm, o_hbm):
        def body(i_vmem, o_vmem):
            pltpu.sync_copy(x_hbm.at[i_vmem.at[0]], o_vmem)

        pltpu.emit_pipeline(
            body,
            grid=(_NIDX // _GW,),
            in_specs=[pl.BlockSpec((1, _GW), lambda i: (0, i))],
            out_specs=[pl.BlockSpec((_GW, 128), lambda i: (i, 0))],
            core_axis_name=("core", "subcore"),
            dimension_semantics=(pltpu.PARALLEL,),
        )(i_hbm, o_hbm)

    return kern(x2d, idx2d)


@jax.jit
def kernel(x, params, sample_idx):
    p = params
    row_ids = jnp.concatenate([sample_idx + b * S for b in range(B)])
    chunk_ids = (row_ids[:, None] * _CHUNKS
                 + jnp.arange(_CHUNKS, dtype=jnp.int32)[None, :])
    xs = _sc_gather(x.reshape(B * S * _CHUNKS, 128),
                    chunk_ids.reshape(1, _NIDX)).reshape(B, SAMPLE, D)

    ctx, parts = pl.pallas_call(
        _attn_kernel,
        grid=(B, S // RQ),
        in_specs=[
            pl.BlockSpec((1, RQ, D), lambda b, i: (b, i, 0)),
            pl.BlockSpec((1, S, D), lambda b, i: (b, 0, 0)),
            pl.BlockSpec((1, SAMPLE, D), lambda b, i: (b, 0, 0)),
            pl.BlockSpec((D, D), lambda b, i: (0, 0)),
            pl.BlockSpec((1, D), lambda b, i: (0, 0)),
            pl.BlockSpec((D, D), lambda b, i: (0, 0)),
            pl.BlockSpec((1, D), lambda b, i: (0, 0)),
            pl.BlockSpec((D, D), lambda b, i: (0, 0)),
            pl.BlockSpec((1, D), lambda b, i: (0, 0)),
        ],
        out_specs=[
            pl.BlockSpec((1, RQ, D), lambda b, i: (b, i, 0)),
            pl.BlockSpec((1, 1, 128), lambda b, i: (b, 0, 0)),
        ],
        out_shape=[
            jax.ShapeDtypeStruct((B, S, D), jnp.float32),
            jax.ShapeDtypeStruct((B, 1, 128), jnp.float32),
        ],
        scratch_shapes=[
            pltpu.VMEM((S, D), jnp.float32),
            pltpu.VMEM((S, D), jnp.float32),
        ],
        compiler_params=pltpu.CompilerParams(
            vmem_limit_bytes=64 * 1024 * 1024),
    )(x, x, xs,
      p["q"]["w"], _row2d(p["q"]["b"]),
      p["k"]["w"], _row2d(p["k"]["b"]),
      p["v"]["w"], _row2d(p["v"]["b"]))

    gate = p["topo_gate"].reshape(1, 1)
    out = pl.pallas_call(
        _ffn_kernel,
        grid=(B, S // RF),
        in_specs=[
            pl.BlockSpec((1, RF, D), lambda b, i: (b, i, 0)),
            pl.BlockSpec((1, RF, D), lambda b, i: (b, i, 0)),
            pl.BlockSpec((1, 1, 128), lambda b, i: (b, 0, 0)),
            pl.BlockSpec((1, 1), lambda b, i: (0, 0)),
            pl.BlockSpec((D, D), lambda b, i: (0, 0)),
            pl.BlockSpec((1, D), lambda b, i: (0, 0)),
            pl.BlockSpec((8, D // 4), lambda b, i: (0, 0)),
            pl.BlockSpec((1, D // 4), lambda b, i: (0, 0)),
            pl.BlockSpec((D // 4, RES), lambda b, i: (0, 0)),
            pl.BlockSpec((1, RES), lambda b, i: (0, 0)),
            pl.BlockSpec((RES, RES), lambda b, i: (0, 0)),
            pl.BlockSpec((1, RES), lambda b, i: (0, 0)),
            pl.BlockSpec((RES, RES), lambda b, i: (0, 0)),
            pl.BlockSpec((1, RES), lambda b, i: (0, 0)),
            pl.BlockSpec((RES, D), lambda b, i: (0, 0)),
            pl.BlockSpec((1, D), lambda b, i: (0, 0)),
            pl.BlockSpec((1, D), lambda b, i: (0, 0)),
            pl.BlockSpec((1, D), lambda b, i: (0, 0)),
            pl.BlockSpec((D, FF), lambda b, i: (0, 0)),
            pl.BlockSpec((1, FF), lambda b, i: (0, 0)),
            pl.BlockSpec((FF, D), lambda b, i: (0, 0)),
            pl.BlockSpec((1, D), lambda b, i: (0, 0)),
            pl.BlockSpec((1, D), lambda b, i: (0, 0)),
            pl.BlockSpec((1, D), lambda b, i: (0, 0)),
        ],
        out_specs=pl.BlockSpec((1, RF, D), lambda b, i: (b, i, 0)),
        out_shape=jax.ShapeDtypeStruct((B, S, D), jnp.float32),
        scratch_shapes=[pltpu.VMEM((1, D), jnp.float32)],
    )(x, ctx, parts, gate,
      p["o"]["w"], _row2d(p["o"]["b"]),
      p["stats1"]["w"], _row2d(p["stats1"]["b"]),
      p["stats2"]["w"], _row2d(p["stats2"]["b"]),
      p["proc0"]["w"], _row2d(p["proc0"]["b"]),
      p["proc1"]["w"], _row2d(p["proc1"]["b"]),
      p["topo_proj"]["w"], _row2d(p["topo_proj"]["b"]),
      _row2d(p["ln1_g"]), _row2d(p["ln1_b"]),
      p["ffn1"]["w"], _row2d(p["ffn1"]["b"]),
      p["ffn2"]["w"], _row2d(p["ffn2"]["b"]),
      _row2d(p["ln2_g"]), _row2d(p["ln2_b"]))
    return out
